# Initial kernel scaffold; baseline (speedup 1.0000x reference)
import jax
import jax.numpy as jnp
from jax.experimental import pallas as pl


def kernel(pos_target, pos_context, neg_context, target_emb, context_emb):
    raise NotImplementedError("write your pallas kernel here")


# speedup vs baseline: 5.3275x; 5.3275x over previous
import functools

import jax
import jax.numpy as jnp
from jax import lax
from jax.experimental import pallas as pl
from jax.experimental.pallas import tpu as pltpu
from jax.experimental.pallas import tpu_sc as plsc

B = 16384
D = 64
NNEG = 20
NW = 32
ROWS_PER_W = B // NW
CB = 64
NCH = ROWS_PER_W // CB
OUTW = 32


def _sc_body(pos_t, pos_c, neg2d, temb, cemb, dots,
             tidx, cidx, nidx, tgtv, ctxv, negv, outv, sem):
    wid = lax.axis_index("s") * 2 + lax.axis_index("c")
    base = wid * ROWS_PER_W
    lane = lax.iota(jnp.int32, 16)
    pltpu.sync_copy(neg2d.at[pl.ds(wid * (ROWS_PER_W * NNEG // 128),
                                   ROWS_PER_W * NNEG // 128)], nidx)

    def chunk_body(ch, carry):
        r0 = base + ch * CB
        pltpu.sync_copy(pos_t.at[pl.ds(r0, CB)], tidx)
        pltpu.sync_copy(pos_c.at[pl.ds(r0, CB)], cidx)

        cps = [pltpu.async_copy(temb.at[tidx], tgtv, sem),
               pltpu.async_copy(cemb.at[cidx], ctxv, sem)]
        for j in range(10):
            cps.append(pltpu.async_copy(cemb.at[nidx.at[ch * 10 + j]],
                                        negv.at[pl.ds(j * 128, 128)], sem))
        for cp in cps:
            cp.wait()

        def row_body(r, c2):
            t0 = tgtv[r, pl.ds(0, 16)]
            t1 = tgtv[r, pl.ds(16, 16)]
            t2 = tgtv[r, pl.ds(32, 16)]
            t3 = tgtv[r, pl.ds(48, 16)]
            p = (t0 * ctxv[r, pl.ds(0, 16)] + t1 * ctxv[r, pl.ds(16, 16)]
                 + t2 * ctxv[r, pl.ds(32, 16)] + t3 * ctxv[r, pl.ds(48, 16)])
            rv0 = jnp.where(lane == 0, jnp.sum(p), 0.0)
            rv1 = jnp.zeros((16,), jnp.float32)
            rn = r * NNEG
            for n in range(NNEG):
                v = (t0 * negv[rn + n, pl.ds(0, 16)]
                     + t1 * negv[rn + n, pl.ds(16, 16)]
                     + t2 * negv[rn + n, pl.ds(32, 16)]
                     + t3 * negv[rn + n, pl.ds(48, 16)])
                s = jnp.sum(v)
                if n < 15:
                    rv0 = jnp.where(lane == 1 + n, s, rv0)
                else:
                    rv1 = jnp.where(lane == n - 15, s, rv1)
            outv[r, pl.ds(0, 16)] = rv0
            outv[r, pl.ds(16, 16)] = rv1
            return c2
        lax.fori_loop(0, CB, row_body, 0)
        pltpu.sync_copy(outv, dots.at[pl.ds(r0, CB)])
        return carry
    lax.fori_loop(0, NCH, chunk_body, 0)


_sc_dots = pl.kernel(
    _sc_body,
    out_type=jax.ShapeDtypeStruct((B, OUTW), jnp.float32),
    mesh=plsc.VectorSubcoreMesh(core_axis_name="c", subcore_axis_name="s"),
    compiler_params=pltpu.CompilerParams(needs_layout_passes=False,
                                         use_tc_tiling_on_sc=False),
    scratch_types=[
        pltpu.VMEM((CB,), jnp.int32),
        pltpu.VMEM((CB,), jnp.int32),
        pltpu.VMEM((ROWS_PER_W * NNEG // 128, 128), jnp.int32),
        pltpu.VMEM((CB, D), jnp.float32),
        pltpu.VMEM((CB, D), jnp.float32),
        pltpu.VMEM((CB * NNEG, D), jnp.float32),
        pltpu.VMEM((CB, OUTW), jnp.float32),
        pltpu.SemaphoreType.DMA,
    ],
)


def _tc_loss_body(d_ref, o_ref):
    x = d_ref[:]
    col = lax.broadcasted_iota(jnp.int32, x.shape, 1) % OUTW
    xc = jnp.clip(x, -10.0, 10.0)
    pos_f = jnp.log1p(jnp.exp(-xc))
    neg_f = jnp.log1p(jnp.exp(xc))
    contrib = jnp.where(col == 0, pos_f,
                        jnp.where(col <= NNEG, neg_f, 0.0))
    o_ref[0, 0] = jnp.sum(contrib) * (1.0 / B)


_tc_loss = pl.pallas_call(
    _tc_loss_body,
    out_shape=jax.ShapeDtypeStruct((1, 1), jnp.float32),
    in_specs=[pl.BlockSpec(memory_space=pltpu.VMEM)],
    out_specs=pl.BlockSpec(memory_space=pltpu.SMEM),
)


def kernel(pos_target, pos_context, neg_context, target_emb, context_emb):
    neg2d = neg_context.reshape(B * NNEG // 128, 128)
    dots = _sc_dots(pos_target, pos_context, neg2d, target_emb, context_emb)
    loss = _tc_loss(dots.reshape(B * OUTW // 128, 128))
    return loss[0, 0]
